# Initial kernel scaffold; baseline (speedup 1.0000x reference)
import jax
import jax.numpy as jnp
from jax.experimental import pallas as pl


def kernel(x, edge_index, W1, b1, g1, be1, W2, b2):
    raise NotImplementedError("write your pallas kernel here")


# speedup vs baseline: 7.5210x; 7.5210x over previous
import functools

import jax
import jax.numpy as jnp
from jax import lax
from jax.experimental import pallas as pl
from jax.experimental.pallas import tpu as pltpu
from jax.experimental.pallas import tpu_sc as plsc

N = 10000
N_PAD = 10240
E = 160000
D = 256
H = 512
EPS = 1e-07

LC = 128
NCHUNK = 4
NSUB = 16
NCORE = 2
EPT = E // NSUB
B = 80
NBATCH = EPT // B
ROWS_PER_TILE = N_PAD // NSUB
IO_ROWS = 128
IO_STEPS = ROWS_PER_TILE // IO_ROWS


def _prep_body(x_ref, t_ref):
    k = pl.program_id(0)
    r = jnp.maximum(x_ref[...], 0.0) + EPS
    p = jnp.exp(r)
    t_ref[...] = jnp.where(k < 2, p, p * r)


def _prep(x):
    RB = 1000
    NB = N // RB
    return pl.pallas_call(
        _prep_body,
        grid=(NCHUNK, NB),
        in_specs=[pl.BlockSpec((RB, LC), lambda k, i: (i, k % 2))],
        out_specs=pl.BlockSpec((RB, LC), lambda k, i: (k * NB + i, 0)),
        out_shape=jax.ShapeDtypeStruct((NCHUNK * N, LC), jnp.float32),
    )(x)


def _sc_body(src_hbm, dst_hbm, table_hbm, zeros_hbm, acc_hbm,
             srcall_v, dstall_v, gidx_v, dstv_v, rows_v, bounce_v, acc_sh, sem):
    c = lax.axis_index("c")
    s = lax.axis_index("s")
    ebase = s * EPT

    pltpu.sync_copy(src_hbm.at[pl.ds(ebase, EPT)], srcall_v)
    pltpu.sync_copy(dst_hbm.at[pl.ds(ebase, EPT)], dstall_v)

    for k in range(2):
        chunk = c * 2 + k

        pltpu.sync_copy(zeros_hbm, bounce_v)
        for t in range(IO_STEPS):
            pltpu.sync_copy(
                bounce_v, acc_sh.at[pl.ds(s * ROWS_PER_TILE + t * IO_ROWS, IO_ROWS)])
        plsc.subcore_barrier()

        def batch_body(b, carry):
            off = b * B
            for j in range(B // 16):
                sl = pl.ds(off + j * 16, 16)
                gidx_v[pl.ds(j * 16, 16)] = srcall_v[sl] + chunk * N
                dstv_v[pl.ds(j * 16, 16)] = dstall_v[sl]
            pltpu.async_copy(table_hbm.at[gidx_v], rows_v, sem).wait()
            pltpu.sync_copy(rows_v, acc_sh.at[dstv_v], add=True)
            return carry

        lax.fori_loop(0, NBATCH, batch_body, 0)
        plsc.subcore_barrier()

        for t in range(IO_STEPS):
            rlo = s * ROWS_PER_TILE + t * IO_ROWS
            pltpu.sync_copy(acc_sh.at[pl.ds(rlo, IO_ROWS)], bounce_v)
            pltpu.sync_copy(bounce_v, acc_hbm.at[pl.ds(chunk * N_PAD + rlo, IO_ROWS)])
        plsc.subcore_barrier()


def _sc_segment_sums(src, dst, table, zeros):
    mesh = plsc.VectorSubcoreMesh(core_axis_name="c", subcore_axis_name="s")
    f = functools.partial(
        pl.kernel,
        out_type=jax.ShapeDtypeStruct((NCHUNK * N_PAD, LC), jnp.float32),
        mesh=mesh,
        scratch_types=[
            pltpu.VMEM((EPT,), jnp.int32),
            pltpu.VMEM((EPT,), jnp.int32),
            pltpu.VMEM((B,), jnp.int32),
            pltpu.VMEM((B,), jnp.int32),
            pltpu.VMEM((B, LC), jnp.float32),
            pltpu.VMEM((IO_ROWS, LC), jnp.float32),
            pltpu.VMEM_SHARED((N_PAD, LC), jnp.float32),
            pltpu.SemaphoreType.DMA,
        ],
    )(_sc_body)
    return f(src, dst, table, zeros)


def _mlp_body(d0_ref, d1_ref, n0_ref, n1_ref, x_ref,
              w1_ref, b1_ref, g1_ref, be1_ref, w2_ref, b2_ref, y_ref):
    den = jnp.concatenate([d0_ref[0], d1_ref[0]], axis=1)
    num = jnp.concatenate([n0_ref[0], n1_ref[0]], axis=1)
    agg = num / (den + 1e-16)
    out = agg + x_ref[...]
    h = jnp.dot(out, w1_ref[...], preferred_element_type=jnp.float32) + b1_ref[...]
    mu = jnp.mean(h, axis=1, keepdims=True)
    var = jnp.mean((h - mu) ** 2, axis=1, keepdims=True)
    h = (h - mu) / jnp.sqrt(var + 1e-5) * g1_ref[...] + be1_ref[...]
    h = jnp.maximum(h, 0.0)
    y_ref[...] = (jnp.dot(h, w2_ref[...], preferred_element_type=jnp.float32)
                  + b2_ref[...])


def _mlp(acc, x, W1, b1, g1, be1, W2, b2):
    RB = 1000
    NB = N // RB
    acc4 = acc.reshape(NCHUNK, N_PAD, LC)
    chunk_spec = lambda k: pl.BlockSpec((1, RB, LC), lambda i, k=k: (k, i, 0))
    return pl.pallas_call(
        _mlp_body,
        grid=(NB,),
        in_specs=[
            chunk_spec(0), chunk_spec(1), chunk_spec(2), chunk_spec(3),
            pl.BlockSpec((RB, D), lambda i: (i, 0)),
            pl.BlockSpec((D, H), lambda i: (0, 0)),
            pl.BlockSpec((1, H), lambda i: (0, 0)),
            pl.BlockSpec((1, H), lambda i: (0, 0)),
            pl.BlockSpec((1, H), lambda i: (0, 0)),
            pl.BlockSpec((H, D), lambda i: (0, 0)),
            pl.BlockSpec((1, D), lambda i: (0, 0)),
        ],
        out_specs=pl.BlockSpec((RB, D), lambda i: (i, 0)),
        out_shape=jax.ShapeDtypeStruct((N, D), jnp.float32),
    )(acc4, acc4, acc4, acc4, x, W1, b1.reshape(1, H), g1.reshape(1, H),
      be1.reshape(1, H), W2, b2.reshape(1, D))


def kernel(x, edge_index, W1, b1, g1, be1, W2, b2):
    src = edge_index[0]
    dst = edge_index[1]
    table = _prep(x)
    zeros = jnp.zeros((IO_ROWS, LC), jnp.float32)
    acc = _sc_segment_sums(src, dst, table, zeros)
    return _mlp(acc, x, W1, b1, g1, be1, W2, b2)


# speedup vs baseline: 11.6755x; 1.5524x over previous
import functools

import jax
import jax.numpy as jnp
from jax import lax
from jax.experimental import pallas as pl
from jax.experimental.pallas import tpu as pltpu
from jax.experimental.pallas import tpu_sc as plsc

N = 10000
N_PAD = 10240
E = 160000
D = 256
H = 512
EPS = 1e-07

LC = 128
NCHUNK = 4
NSUB = 16
NCORE = 2
EPT = E // NSUB
B = 80
NBATCH = EPT // B
ROWS_PER_TILE = N_PAD // NSUB
IO_ROWS = B
IO_STEPS = ROWS_PER_TILE // IO_ROWS


def _prep_body(x_ref, t_ref):
    k = pl.program_id(0)
    r = jnp.maximum(x_ref[...], 0.0) + EPS
    p = jnp.exp(r)
    t_ref[...] = jnp.where(k < 2, p, p * r)


def _prep(x):
    RB = 1000
    NB = N // RB
    return pl.pallas_call(
        _prep_body,
        grid=(NCHUNK, NB),
        in_specs=[pl.BlockSpec((RB, LC), lambda k, i: (i, k % 2))],
        out_specs=pl.BlockSpec((RB, LC), lambda k, i: (k * NB + i, 0)),
        out_shape=jax.ShapeDtypeStruct((NCHUNK * N, LC), jnp.float32),
    )(x)


def _sc_body(gidx4_hbm, dst3_hbm, table_hbm, zeros_hbm, acc_hbm,
             gidxall_v, dstall_v, rows_a, rows_b, acc_sh, sem_a, sem_b):
    c = lax.axis_index("c")
    s = lax.axis_index("s")
    ebase = s * EPT

    pltpu.sync_copy(dst3_hbm.at[s], dstall_v)

    for k in range(2):
        chunk = c * 2 + k

        pltpu.sync_copy(zeros_hbm, rows_a)
        for t in range(IO_STEPS):
            pltpu.sync_copy(
                rows_a, acc_sh.at[pl.ds(s * ROWS_PER_TILE + t * IO_ROWS, IO_ROWS)])
        pltpu.sync_copy(gidx4_hbm.at[pl.ds(chunk * E + ebase, EPT)], gidxall_v)
        plsc.subcore_barrier()

        def gather(b, rows_v, sem):
            return pltpu.async_copy(
                table_hbm.at[gidxall_v.at[pl.ds(b * B, B)]], rows_v, sem)

        def scatter(b, rows_v):
            pltpu.sync_copy(rows_v, acc_sh.at[dstall_v.at[b]], add=True)

        gather(0, rows_a, sem_a)

        def batch2_body(bb, carry):
            b0 = bb * 2
            gather(b0 + 1, rows_b, sem_b)
            pltpu.make_async_copy(zeros_hbm, rows_a, sem_a).wait()
            scatter(b0, rows_a)
            gather(b0 + 2, rows_a, sem_a)
            pltpu.make_async_copy(zeros_hbm, rows_b, sem_b).wait()
            scatter(b0 + 1, rows_b)
            return carry

        lax.fori_loop(0, (NBATCH - 1) // 2, batch2_body, 0)
        pltpu.make_async_copy(zeros_hbm, rows_a, sem_a).wait()
        scatter(NBATCH - 1, rows_a)
        plsc.subcore_barrier()

        for t in range(IO_STEPS):
            rlo = s * ROWS_PER_TILE + t * IO_ROWS
            pltpu.sync_copy(acc_sh.at[pl.ds(rlo, IO_ROWS)], rows_a)
            pltpu.sync_copy(rows_a, acc_hbm.at[pl.ds(chunk * N_PAD + rlo, IO_ROWS)])
        plsc.subcore_barrier()


def _sc_segment_sums(gidx4, dst3, table, zeros):
    mesh = plsc.VectorSubcoreMesh(core_axis_name="c", subcore_axis_name="s")
    f = functools.partial(
        pl.kernel,
        out_type=jax.ShapeDtypeStruct((NCHUNK * N_PAD, LC), jnp.float32),
        mesh=mesh,
        scratch_types=[
            pltpu.VMEM((EPT,), jnp.int32),
            pltpu.VMEM((NBATCH, B), jnp.int32),
            pltpu.VMEM((B, LC), jnp.float32),
            pltpu.VMEM((B, LC), jnp.float32),
            pltpu.VMEM_SHARED((N_PAD, LC), jnp.float32),
            pltpu.SemaphoreType.DMA,
            pltpu.SemaphoreType.DMA,
        ],
    )(_sc_body)
    return f(gidx4, dst3, table, zeros)


def _mlp_body(d0_ref, d1_ref, n0_ref, n1_ref, x_ref,
              w1_ref, b1_ref, g1_ref, be1_ref, w2_ref, b2_ref, y_ref):
    den = jnp.concatenate([d0_ref[0], d1_ref[0]], axis=1)
    num = jnp.concatenate([n0_ref[0], n1_ref[0]], axis=1)
    agg = num / (den + 1e-16)
    out = agg + x_ref[...]
    h = jnp.dot(out, w1_ref[...], preferred_element_type=jnp.float32) + b1_ref[...]
    mu = jnp.mean(h, axis=1, keepdims=True)
    var = jnp.mean((h - mu) ** 2, axis=1, keepdims=True)
    h = (h - mu) / jnp.sqrt(var + 1e-5) * g1_ref[...] + be1_ref[...]
    h = jnp.maximum(h, 0.0)
    y_ref[...] = (jnp.dot(h, w2_ref[...], preferred_element_type=jnp.float32)
                  + b2_ref[...])


def _mlp(acc, x, W1, b1, g1, be1, W2, b2):
    RB = 1000
    NB = N // RB
    acc4 = acc.reshape(NCHUNK, N_PAD, LC)
    chunk_spec = lambda k: pl.BlockSpec((1, RB, LC), lambda i, k=k: (k, i, 0))
    return pl.pallas_call(
        _mlp_body,
        grid=(NB,),
        in_specs=[
            chunk_spec(0), chunk_spec(1), chunk_spec(2), chunk_spec(3),
            pl.BlockSpec((RB, D), lambda i: (i, 0)),
            pl.BlockSpec((D, H), lambda i: (0, 0)),
            pl.BlockSpec((1, H), lambda i: (0, 0)),
            pl.BlockSpec((1, H), lambda i: (0, 0)),
            pl.BlockSpec((1, H), lambda i: (0, 0)),
            pl.BlockSpec((H, D), lambda i: (0, 0)),
            pl.BlockSpec((1, D), lambda i: (0, 0)),
        ],
        out_specs=pl.BlockSpec((RB, D), lambda i: (i, 0)),
        out_shape=jax.ShapeDtypeStruct((N, D), jnp.float32),
    )(acc4, acc4, acc4, acc4, x, W1, b1.reshape(1, H), g1.reshape(1, H),
      be1.reshape(1, H), W2, b2.reshape(1, D))


def kernel(x, edge_index, W1, b1, g1, be1, W2, b2):
    src = edge_index[0]
    dst = edge_index[1]
    gidx4 = (src[None, :] + (jnp.arange(NCHUNK, dtype=jnp.int32) * N)[:, None]
             ).reshape(NCHUNK * E)
    dst3 = dst.reshape(NSUB, NBATCH, B)
    table = _prep(x)
    zeros = jnp.zeros((IO_ROWS, LC), jnp.float32)
    acc = _sc_segment_sums(gidx4, dst3, table, zeros)
    return _mlp(acc, x, W1, b1, g1, be1, W2, b2)


# speedup vs baseline: 13.7259x; 1.1756x over previous
import functools

import jax
import jax.numpy as jnp
from jax import lax
from jax.experimental import pallas as pl
from jax.experimental.pallas import tpu as pltpu
from jax.experimental.pallas import tpu_sc as plsc

N = 10000
N_PAD = 10240
E = 160000
D = 256
H = 512
EPS = 1e-07

LC = 128
NCHUNK = 4
NSUB = 16
NCORE = 2
EPT = E // NSUB
B = 80
NBATCH = EPT // B
NBUF = 3
ROWS_PER_TILE = N_PAD // NSUB
IO_ROWS = B
IO_STEPS = ROWS_PER_TILE // IO_ROWS


def _prep_body(x_ref, t_ref):
    k = pl.program_id(0)
    r = jnp.maximum(x_ref[...], 0.0) + EPS
    p = jnp.exp(r)
    t_ref[...] = jnp.where(k < 2, p, p * r)


def _prep(x):
    RB = 1000
    NB = N // RB
    return pl.pallas_call(
        _prep_body,
        grid=(NCHUNK, NB),
        in_specs=[pl.BlockSpec((RB, LC), lambda k, i: (i, k % 2))],
        out_specs=pl.BlockSpec((RB, LC), lambda k, i: (k * NB + i, 0)),
        out_shape=jax.ShapeDtypeStruct((NCHUNK * N, LC), jnp.float32),
    )(x)


def _sc_body(gidx4_hbm, dst_hbm, table_hbm, zeros_hbm, acc_hbm,
             gidxall_v, r0, r1, r2, d0, d1, d2,
             acc_sh, s0, s1, s2, ds0, ds1, ds2):
    c = lax.axis_index("c")
    s = lax.axis_index("s")
    ebase = s * EPT
    bufs = [r0, r1, r2]
    dbufs = [d0, d1, d2]
    sems = [s0, s1, s2]
    dsems = [ds0, ds1, ds2]

    for k in range(2):
        chunk = c * 2 + k

        pltpu.sync_copy(zeros_hbm, r0)
        for t in range(IO_STEPS):
            pltpu.sync_copy(
                r0, acc_sh.at[pl.ds(s * ROWS_PER_TILE + t * IO_ROWS, IO_ROWS)])
        pltpu.sync_copy(gidx4_hbm.at[pl.ds(chunk * E + ebase, EPT)], gidxall_v)
        plsc.subcore_barrier()

        def gather(b, i):
            pltpu.async_copy(dst_hbm.at[pl.ds(ebase + b * B, B)],
                             dbufs[i], dsems[i])
            pltpu.async_copy(
                table_hbm.at[gidxall_v.at[pl.ds(b * B, B)]], bufs[i], sems[i])

        def drain_scatter(b, i):
            pltpu.make_async_copy(dst_hbm.at[pl.ds(0, B)],
                                  dbufs[i], dsems[i]).wait()
            pltpu.make_async_copy(zeros_hbm, bufs[i], sems[i]).wait()
            pltpu.sync_copy(bufs[i], acc_sh.at[dbufs[i]], add=True)

        for j in range(NBUF - 1):
            gather(j, j)

        def ring_body(bb, carry):
            base = bb * NBUF
            for j in range(NBUF):
                gather(base + j + NBUF - 1, (j + NBUF - 1) % NBUF)
                drain_scatter(base + j, j)
            return carry

        lax.fori_loop(0, (NBATCH - (NBUF - 1)) // NBUF, ring_body, 0)
        for j in range(NBUF - 1):
            drain_scatter(NBATCH - (NBUF - 1) + j, j)
        plsc.subcore_barrier()

        for t in range(IO_STEPS):
            rlo = s * ROWS_PER_TILE + t * IO_ROWS
            pltpu.sync_copy(acc_sh.at[pl.ds(rlo, IO_ROWS)], r0)
            pltpu.sync_copy(r0, acc_hbm.at[pl.ds(chunk * N_PAD + rlo, IO_ROWS)])
        plsc.subcore_barrier()


def _sc_segment_sums(gidx4, dst, table, zeros):
    mesh = plsc.VectorSubcoreMesh(core_axis_name="c", subcore_axis_name="s")
    f = functools.partial(
        pl.kernel,
        out_type=jax.ShapeDtypeStruct((NCHUNK * N_PAD, LC), jnp.float32),
        mesh=mesh,
        scratch_types=[
            pltpu.VMEM((EPT,), jnp.int32),
            pltpu.VMEM((B, LC), jnp.float32),
            pltpu.VMEM((B, LC), jnp.float32),
            pltpu.VMEM((B, LC), jnp.float32),
            pltpu.VMEM((B,), jnp.int32),
            pltpu.VMEM((B,), jnp.int32),
            pltpu.VMEM((B,), jnp.int32),
            pltpu.VMEM_SHARED((N_PAD, LC), jnp.float32),
            pltpu.SemaphoreType.DMA,
            pltpu.SemaphoreType.DMA,
            pltpu.SemaphoreType.DMA,
            pltpu.SemaphoreType.DMA,
            pltpu.SemaphoreType.DMA,
            pltpu.SemaphoreType.DMA,
        ],
    )(_sc_body)
    return f(gidx4, dst, table, zeros)


def _mlp_body(d0_ref, d1_ref, n0_ref, n1_ref, x_ref,
              w1_ref, b1_ref, g1_ref, be1_ref, w2_ref, b2_ref, y_ref):
    den = jnp.concatenate([d0_ref[0], d1_ref[0]], axis=1)
    num = jnp.concatenate([n0_ref[0], n1_ref[0]], axis=1)
    agg = num / (den + 1e-16)
    out = agg + x_ref[...]
    h = jnp.dot(out, w1_ref[...], preferred_element_type=jnp.float32) + b1_ref[...]
    mu = jnp.mean(h, axis=1, keepdims=True)
    var = jnp.mean((h - mu) ** 2, axis=1, keepdims=True)
    h = (h - mu) / jnp.sqrt(var + 1e-5) * g1_ref[...] + be1_ref[...]
    h = jnp.maximum(h, 0.0)
    y_ref[...] = (jnp.dot(h, w2_ref[...], preferred_element_type=jnp.float32)
                  + b2_ref[...])


def _mlp(acc, x, W1, b1, g1, be1, W2, b2):
    RB = 1000
    NB = N // RB
    acc4 = acc.reshape(NCHUNK, N_PAD, LC)
    chunk_spec = lambda k: pl.BlockSpec((1, RB, LC), lambda i, k=k: (k, i, 0))
    return pl.pallas_call(
        _mlp_body,
        grid=(NB,),
        in_specs=[
            chunk_spec(0), chunk_spec(1), chunk_spec(2), chunk_spec(3),
            pl.BlockSpec((RB, D), lambda i: (i, 0)),
            pl.BlockSpec((D, H), lambda i: (0, 0)),
            pl.BlockSpec((1, H), lambda i: (0, 0)),
            pl.BlockSpec((1, H), lambda i: (0, 0)),
            pl.BlockSpec((1, H), lambda i: (0, 0)),
            pl.BlockSpec((H, D), lambda i: (0, 0)),
            pl.BlockSpec((1, D), lambda i: (0, 0)),
        ],
        out_specs=pl.BlockSpec((RB, D), lambda i: (i, 0)),
        out_shape=jax.ShapeDtypeStruct((N, D), jnp.float32),
    )(acc4, acc4, acc4, acc4, x, W1, b1.reshape(1, H), g1.reshape(1, H),
      be1.reshape(1, H), W2, b2.reshape(1, D))


def kernel(x, edge_index, W1, b1, g1, be1, W2, b2):
    src = edge_index[0]
    dst = edge_index[1]
    gidx4 = (src[None, :] + (jnp.arange(NCHUNK, dtype=jnp.int32) * N)[:, None]
             ).reshape(NCHUNK * E)
    table = _prep(x)
    zeros = jnp.zeros((IO_ROWS, LC), jnp.float32)
    acc = _sc_segment_sums(gidx4, dst, table, zeros)
    return _mlp(acc, x, W1, b1, g1, be1, W2, b2)


# speedup vs baseline: 14.3515x; 1.0456x over previous
import functools

import jax
import jax.numpy as jnp
from jax import lax
from jax.experimental import pallas as pl
from jax.experimental.pallas import tpu as pltpu
from jax.experimental.pallas import tpu_sc as plsc

N = 10000
N_PAD = 10240
E = 160000
D = 256
H = 512
EPS = 1e-07

LC = 128
NCHUNK = 4
NSUB = 16
NCORE = 2
EPT = E // NSUB
B = 80
NBATCH = EPT // B
NBUF = 3
ROWS_PER_TILE = N_PAD // NSUB
IO_ROWS = B
IO_STEPS = ROWS_PER_TILE // IO_ROWS


def _prep_body(x_ref, p_ref, q_ref):
    r = jnp.maximum(x_ref[...], 0.0) + EPS
    p = jnp.exp(r)
    p_ref[...] = p
    q_ref[...] = p * r


def _prep(x):
    RB = 1000
    NB = N // RB
    spec = pl.BlockSpec((RB, LC), lambda c, i: (c * NB + i, 0))
    return pl.pallas_call(
        _prep_body,
        grid=(2, NB),
        in_specs=[pl.BlockSpec((RB, LC), lambda c, i: (i, c))],
        out_specs=(spec, spec),
        out_shape=(jax.ShapeDtypeStruct((2 * N, LC), jnp.float32),
                   jax.ShapeDtypeStruct((2 * N, LC), jnp.float32)),
    )(x)


def _sc_body(gidx2_hbm, dst_hbm, p_hbm, q_hbm, zeros_hbm, acc_hbm,
             gidxall_v, r0, r1, r2, d0, d1, d2,
             acc_sh, s0, s1, s2, ds0, ds1, ds2):
    c = lax.axis_index("c")
    s = lax.axis_index("s")
    ebase = s * EPT
    bufs = [r0, r1, r2]
    dbufs = [d0, d1, d2]
    sems = [s0, s1, s2]
    dsems = [ds0, ds1, ds2]

    pltpu.sync_copy(gidx2_hbm.at[pl.ds(c * E + ebase, EPT)], gidxall_v)

    for k in range(2):
        table_hbm = (p_hbm, q_hbm)[k]

        pltpu.sync_copy(zeros_hbm, r0)
        for t in range(IO_STEPS):
            pltpu.sync_copy(
                r0, acc_sh.at[pl.ds(s * ROWS_PER_TILE + t * IO_ROWS, IO_ROWS)])
        plsc.subcore_barrier()

        def gather(b, i):
            pltpu.async_copy(dst_hbm.at[pl.ds(ebase + b * B, B)],
                             dbufs[i], dsems[i])
            pltpu.async_copy(
                table_hbm.at[gidxall_v.at[pl.ds(b * B, B)]], bufs[i], sems[i])

        def drain_scatter(b, i):
            pltpu.make_async_copy(dst_hbm.at[pl.ds(0, B)],
                                  dbufs[i], dsems[i]).wait()
            pltpu.make_async_copy(zeros_hbm, bufs[i], sems[i]).wait()
            pltpu.sync_copy(bufs[i], acc_sh.at[dbufs[i]], add=True)

        for j in range(NBUF - 1):
            gather(j, j)

        def ring_body(bb, carry):
            base = bb * NBUF
            for j in range(NBUF):
                gather(base + j + NBUF - 1, (j + NBUF - 1) % NBUF)
                drain_scatter(base + j, j)
            return carry

        lax.fori_loop(0, (NBATCH - (NBUF - 1)) // NBUF, ring_body, 0)
        for j in range(NBUF - 1):
            drain_scatter(NBATCH - (NBUF - 1) + j, j)
        plsc.subcore_barrier()

        for t in range(IO_STEPS):
            rlo = s * ROWS_PER_TILE + t * IO_ROWS
            pltpu.sync_copy(acc_sh.at[pl.ds(rlo, IO_ROWS)], r0)
            pltpu.sync_copy(
                r0, acc_hbm.at[pl.ds((k * 2 + c) * N_PAD + rlo, IO_ROWS)])
        plsc.subcore_barrier()


def _sc_segment_sums(gidx2, dst, p_tab, q_tab, zeros):
    mesh = plsc.VectorSubcoreMesh(core_axis_name="c", subcore_axis_name="s")
    f = functools.partial(
        pl.kernel,
        out_type=jax.ShapeDtypeStruct((NCHUNK * N_PAD, LC), jnp.float32),
        mesh=mesh,
        scratch_types=[
            pltpu.VMEM((EPT,), jnp.int32),
            pltpu.VMEM((B, LC), jnp.float32),
            pltpu.VMEM((B, LC), jnp.float32),
            pltpu.VMEM((B, LC), jnp.float32),
            pltpu.VMEM((B,), jnp.int32),
            pltpu.VMEM((B,), jnp.int32),
            pltpu.VMEM((B,), jnp.int32),
            pltpu.VMEM_SHARED((N_PAD, LC), jnp.float32),
            pltpu.SemaphoreType.DMA,
            pltpu.SemaphoreType.DMA,
            pltpu.SemaphoreType.DMA,
            pltpu.SemaphoreType.DMA,
            pltpu.SemaphoreType.DMA,
            pltpu.SemaphoreType.DMA,
        ],
    )(_sc_body)
    return f(gidx2, dst, p_tab, q_tab, zeros)


def _mlp_body(d0_ref, d1_ref, n0_ref, n1_ref, x_ref,
              w1_ref, b1_ref, g1_ref, be1_ref, w2_ref, b2_ref, y_ref):
    den = jnp.concatenate([d0_ref[0], d1_ref[0]], axis=1)
    num = jnp.concatenate([n0_ref[0], n1_ref[0]], axis=1)
    agg = num / (den + 1e-16)
    out = agg + x_ref[...]
    h = jnp.dot(out.astype(jnp.bfloat16), w1_ref[...].astype(jnp.bfloat16),
                preferred_element_type=jnp.float32) + b1_ref[...]
    mu = jnp.mean(h, axis=1, keepdims=True)
    var = jnp.mean((h - mu) ** 2, axis=1, keepdims=True)
    h = (h - mu) / jnp.sqrt(var + 1e-5) * g1_ref[...] + be1_ref[...]
    h = jnp.maximum(h, 0.0)
    y_ref[...] = (jnp.dot(h.astype(jnp.bfloat16), w2_ref[...].astype(jnp.bfloat16),
                          preferred_element_type=jnp.float32) + b2_ref[...])


def _mlp(acc, x, W1, b1, g1, be1, W2, b2):
    RB = 1000
    NB = N // RB
    acc4 = acc.reshape(NCHUNK, N_PAD, LC)
    chunk_spec = lambda k: pl.BlockSpec((1, RB, LC), lambda i, k=k: (k, i, 0))
    return pl.pallas_call(
        _mlp_body,
        grid=(NB,),
        in_specs=[
            chunk_spec(0), chunk_spec(1), chunk_spec(2), chunk_spec(3),
            pl.BlockSpec((RB, D), lambda i: (i, 0)),
            pl.BlockSpec((D, H), lambda i: (0, 0)),
            pl.BlockSpec((1, H), lambda i: (0, 0)),
            pl.BlockSpec((1, H), lambda i: (0, 0)),
            pl.BlockSpec((1, H), lambda i: (0, 0)),
            pl.BlockSpec((H, D), lambda i: (0, 0)),
            pl.BlockSpec((1, D), lambda i: (0, 0)),
        ],
        out_specs=pl.BlockSpec((RB, D), lambda i: (i, 0)),
        out_shape=jax.ShapeDtypeStruct((N, D), jnp.float32),
    )(acc4, acc4, acc4, acc4, x, W1, b1.reshape(1, H), g1.reshape(1, H),
      be1.reshape(1, H), W2, b2.reshape(1, D))


def kernel(x, edge_index, W1, b1, g1, be1, W2, b2):
    src = edge_index[0]
    dst = edge_index[1]
    gidx2 = (src[None, :] + (jnp.arange(2, dtype=jnp.int32) * N)[:, None]
             ).reshape(2 * E)
    p_tab, q_tab = _prep(x)
    zeros = jnp.zeros((IO_ROWS, LC), jnp.float32)
    acc = _sc_segment_sums(gidx2, dst, p_tab, q_tab, zeros)
    return _mlp(acc, x, W1, b1, g1, be1, W2, b2)
